# baseline (speedup 1.0000x reference)
import jax
import jax.numpy as jnp
from jax import lax
from jax.experimental import pallas as pl
from jax.experimental.pallas import tpu as pltpu
from jax.experimental.pallas import tpu_sc as plsc

D = 64
L = 16
R = D // L
B = 4096
H = 50
H_PAD = 56
T = 100
T_PAD = 104
G = 0.5
GROUPS = (0, 16, 32, 48, 64, 80, 88)

_INFO = plsc.get_sparse_core_info()
NC, NS = _INFO.num_cores, _INFO.num_subcores
NW = NC * NS
BPW = B // NW


def _lanesum(v, lanes):
    for s in (8, 4, 2, 1):
        v = v + v.at[lanes ^ s].get(mode="promise_in_bounds")
    return v


def _rsqrt(x):
    i = lax.bitcast_convert_type(x, jnp.int32)
    y = lax.bitcast_convert_type(jnp.int32(0x5F3759DF) - (i >> 1), jnp.float32)
    for _ in range(3):
        y = y * (1.5 - 0.5 * x * y * y)
    return y


def _body(ui_hbm, ii_hbm, ti_hbm, ue_hbm, ie_hbm, wt_hbm, out_hbm,
          ii_v, ti_v, ui_v, wt_v, urows_v, hist_v, tgt_v,
          ssq_tr, dot_tr, out_v, sem_u, sem_h, sem_t):
    wid = lax.axis_index("s") * NC + lax.axis_index("c")
    base = wid * BPW
    pltpu.sync_copy(ii_hbm.at[pl.ds(base, BPW)], ii_v)
    pltpu.sync_copy(ti_hbm.at[pl.ds(base, BPW)], ti_v)
    pltpu.sync_copy(ui_hbm.at[pl.ds(base, BPW)], ui_v)
    pltpu.sync_copy(wt_hbm, wt_v)
    pltpu.async_copy(ue_hbm.at[ui_v], urows_v, sem_u).wait()

    lanes = lax.iota(jnp.int32, L)
    lanes16 = lanes * L

    def user_body(b, carry):
        ch = pltpu.async_copy(ie_hbm.at[ii_v.at[b]], hist_v, sem_h)
        ct = pltpu.async_copy(ie_hbm.at[ti_v.at[b]], tgt_v, sem_t)
        ch.wait()

        cacc = jnp.where(ii_v[b, pl.ds(0, L)] != 0, 1.0, 0.0)
        cacc = cacc + jnp.where(ii_v[b, pl.ds(L, L)] != 0, 1.0, 0.0)
        cacc = cacc + jnp.where(ii_v[b, pl.ds(2 * L, L)] != 0, 1.0, 0.0)
        tail = ii_v[b, pl.ds(40, L)]
        cacc = cacc + jnp.where((lanes >= 8) & (tail != 0), 1.0, 0.0)
        inv = 1.0 / _lanesum(cacc, lanes)

        accs = [jnp.zeros((L,), jnp.float32) for _ in range(R)]
        for j in range(H_PAD):
            for r in range(R):
                accs[r] = accs[r] + hist_v[j, pl.ds(L * r, L)]

        hs = [jnp.zeros((L,), jnp.float32) for _ in range(R)]
        for q in range(R):
            pr = accs[q] * inv
            for l in range(L):
                p = pr.at[jnp.full((L,), l, jnp.int32)].get(
                    mode="promise_in_bounds")
                k = L * q + l
                for r in range(R):
                    hs[r] = hs[r] + p * wt_v[k, pl.ds(L * r, L)]

        uv = [G * urows_v[b, pl.ds(L * r, L)] + (1.0 - G) * hs[r]
              for r in range(R)]
        un = uv[0] * uv[0]
        for r in range(1, R):
            un = un + uv[r] * uv[r]
        ussq = _lanesum(un, lanes)

        ct.wait()
        for n0 in GROUPS:
            for j in range(L):
                n = n0 + j
                t = [tgt_v[n, pl.ds(L * r, L)] for r in range(R)]
                sv = t[0] * t[0]
                dv = t[0] * uv[0]
                for r in range(1, R):
                    sv = sv + t[r] * t[r]
                    dv = dv + t[r] * uv[r]
                plsc.store_scatter(ssq_tr, [lanes16 + j], sv)
                plsc.store_scatter(dot_tr, [lanes16 + j], dv)
            ssqv = ssq_tr[pl.ds(0, L)]
            dotv = dot_tr[pl.ds(0, L)]
            for l in range(1, L):
                ssqv = ssqv + ssq_tr[pl.ds(L * l, L)]
                dotv = dotv + dot_tr[pl.ds(L * l, L)]
            x = jnp.maximum(ssqv * ussq, 1e-30)
            out_v[b, pl.ds(n0, L)] = dotv * _rsqrt(x)
        return carry

    lax.fori_loop(0, BPW, user_body, 0)
    pltpu.sync_copy(out_v, out_hbm.at[pl.ds(base, BPW)])


def kernel(user_idx, interacted_items, target_idx, user_emb, item_emb, W):
    ui = user_idx.astype(jnp.int32)
    ii = jnp.pad(interacted_items.astype(jnp.int32), ((0, 0), (0, H_PAD - H)))
    ti = jnp.pad(target_idx.astype(jnp.int32), ((0, 0), (0, T_PAD - T)))
    wt = W.T.astype(jnp.float32)
    mesh = plsc.VectorSubcoreMesh(core_axis_name="c", subcore_axis_name="s")
    run = pl.kernel(
        _body,
        out_type=jax.ShapeDtypeStruct((B, T_PAD), jnp.float32),
        mesh=mesh,
        compiler_params=pltpu.CompilerParams(needs_layout_passes=False,
                                             use_tc_tiling_on_sc=False),
        scratch_types=[
            pltpu.VMEM((BPW, H_PAD), jnp.int32),
            pltpu.VMEM((BPW, T_PAD), jnp.int32),
            pltpu.VMEM((BPW,), jnp.int32),
            pltpu.VMEM((D, D), jnp.float32),
            pltpu.VMEM((BPW, D), jnp.float32),
            pltpu.VMEM((H_PAD, D), jnp.float32),
            pltpu.VMEM((T_PAD, D), jnp.float32),
            pltpu.VMEM((L * L,), jnp.float32),
            pltpu.VMEM((L * L,), jnp.float32),
            pltpu.VMEM((BPW, T_PAD), jnp.float32),
            pltpu.SemaphoreType.DMA,
            pltpu.SemaphoreType.DMA,
            pltpu.SemaphoreType.DMA,
        ],
    )
    out = run(ui, ii, ti, user_emb.astype(jnp.float32),
              item_emb.astype(jnp.float32), wt)
    return out[:, :T]
